# Initial kernel scaffold; baseline (speedup 1.0000x reference)
import jax
import jax.numpy as jnp
from jax.experimental import pallas as pl


def kernel(x, edge_index, edge_weight, W, b):
    raise NotImplementedError("write your pallas kernel here")


# speedup vs baseline: 18.3619x; 18.3619x over previous
import functools

import jax
import jax.numpy as jnp
from jax import lax
from jax.experimental import pallas as pl
from jax.experimental.pallas import tpu as pltpu
from jax.experimental.pallas import tpu_sc as plsc

NC = 2
NS = 16
NW = NC * NS
LANES = 16


def _sigmoid(v):
    return 1.0 / (1.0 + jnp.exp(-v))


def kernel(x, edge_index, edge_weight, W, b):
    N, F_in = x.shape
    F_out = W.shape[0]
    E = edge_weight.shape[0]

    E_per = E // NW
    n_pad = -(-N // 1024) * 1024
    rows_per_tile = n_pad // NS
    ZR = 64
    nz = rows_per_tile // ZR
    CH = 80
    n_ch = E_per // CH
    FL = F_out // LANES

    mesh = plsc.VectorSubcoreMesh(core_axis_name="c", subcore_axis_name="s")

    @functools.partial(
        pl.kernel, mesh=mesh,
        compiler_params=pltpu.CompilerParams(needs_layout_passes=False),
        out_type=jax.ShapeDtypeStruct((NW, n_pad), jnp.float32),
        scratch_types=[
            pltpu.VMEM((n_pad,), jnp.float32),
            pltpu.VMEM((E_per,), jnp.int32),
            pltpu.VMEM((E_per,), jnp.float32),
        ],
    )
    def deg_kernel(col_hbm, ew_hbm, deg_hbm, deg_v, col_v, ewv):
        cid = lax.axis_index("c")
        sid = lax.axis_index("s")
        wid = sid * NC + cid
        base = wid * E_per
        zero16 = jnp.zeros((LANES,), jnp.float32)

        def zb(i, carry):
            deg_v[pl.ds(i * LANES, LANES)] = zero16
            return carry
        lax.fori_loop(0, n_pad // LANES, zb, 0)

        pltpu.sync_copy(col_hbm.at[pl.ds(base, E_per)], col_v)
        pltpu.sync_copy(ew_hbm.at[pl.ds(base, E_per)], ewv)

        def acc(i, carry):
            idx = col_v[pl.ds(i * LANES, LANES)]
            s = _sigmoid(ewv[pl.ds(i * LANES, LANES)])
            plsc.addupdate_scatter(deg_v, [idx], s)
            return carry
        lax.fori_loop(0, E_per // LANES, acc, 0)

        pltpu.sync_copy(deg_v, deg_hbm.at[wid])

    row_ids = edge_index[0]
    col_ids = edge_index[1]
    degp = deg_kernel(col_ids, edge_weight)

    BN = 256
    nb = n_pad // BN
    x_pad = jnp.pad(x, ((0, n_pad - N), (0, 0)))

    def g_body(x_ref, w_ref, dp_ref, g_ref):
        h = lax.dot_general(x_ref[...], w_ref[...], (((1,), (1,)), ((), ())),
                            preferred_element_type=jnp.float32)
        deg = jnp.sum(dp_ref[...], axis=0) + 1.0
        dinv = lax.rsqrt(deg)
        g_ref[...] = h * dinv[:, None]

    g = pl.pallas_call(
        g_body,
        grid=(nb,),
        in_specs=[
            pl.BlockSpec((BN, F_in), lambda i: (i, 0)),
            pl.BlockSpec((F_out, F_in), lambda i: (0, 0)),
            pl.BlockSpec((NW, BN), lambda i: (0, i)),
        ],
        out_specs=pl.BlockSpec((BN, F_out), lambda i: (i, 0)),
        out_shape=jax.ShapeDtypeStruct((n_pad, F_out), jnp.float32),
    )(x_pad, W, degp)

    @functools.partial(
        pl.kernel, mesh=mesh,
        compiler_params=pltpu.CompilerParams(needs_layout_passes=False),
        out_type=jax.ShapeDtypeStruct((NC, n_pad, F_out), jnp.float32),
        scratch_types=[
            pltpu.VMEM((E_per,), jnp.int32),
            pltpu.VMEM((E_per,), jnp.float32),
            pltpu.VMEM((CH,), jnp.int32),
            pltpu.VMEM((CH, 128), jnp.float32),
            pltpu.VMEM((ZR, 128), jnp.float32),
            pltpu.VMEM_SHARED((n_pad, 128), jnp.float32),
            pltpu.SemaphoreType.DMA,
        ],
    )
    def edge_kernel(g_hbm, row_hbm, col_hbm, ew_hbm, acc_hbm,
                    row_v, ewv, col_v, rows_v, zbuf, acc_sh, sem):
        cid = lax.axis_index("c")
        sid = lax.axis_index("s")
        wid = sid * NC + cid
        base = wid * E_per
        zero16 = jnp.zeros((LANES,), jnp.float32)

        def zb(i, carry):
            zbuf[i // FL, pl.ds((i % FL) * LANES, LANES)] = zero16
            return carry
        lax.fori_loop(0, ZR * FL, zb, 0)

        def zacc(t, carry):
            pltpu.sync_copy(
                zbuf, acc_sh.at[pl.ds(sid * rows_per_tile + t * ZR, ZR)])
            return carry
        lax.fori_loop(0, nz, zacc, 0)

        plsc.subcore_barrier()

        pltpu.sync_copy(row_hbm.at[pl.ds(base, E_per)], row_v)
        pltpu.sync_copy(ew_hbm.at[pl.ds(base, E_per)], ewv)

        def chunk(ci, carry):
            off = ci * CH
            pltpu.sync_copy(col_hbm.at[pl.ds(base + off, CH)], col_v)
            pltpu.async_copy(g_hbm.at[row_v.at[pl.ds(off, CH)]],
                             rows_v, sem).wait()

            def scale16(gi, c2):
                sv = _sigmoid(ewv[pl.ds(off + gi * LANES, LANES)])
                for k in range(LANES):
                    e = gi * LANES + k
                    s = sv[k]
                    for j in range(FL):
                        sl = pl.ds(j * LANES, LANES)
                        rows_v[e, sl] = rows_v[e, sl] * s
                return c2
            lax.fori_loop(0, CH // LANES, scale16, 0)

            pltpu.sync_copy(rows_v, acc_sh.at[col_v], add=True)
            return carry
        lax.fori_loop(0, n_ch, chunk, 0)

        plsc.subcore_barrier()
        pltpu.sync_copy(
            acc_sh.at[pl.ds(sid * rows_per_tile, rows_per_tile)],
            acc_hbm.at[cid, pl.ds(sid * rows_per_tile, rows_per_tile)])

    accs = edge_kernel(g, row_ids, col_ids, edge_weight)

    def out_body(a_ref, g_ref, dp_ref, b_ref, o_ref):
        deg = jnp.sum(dp_ref[...], axis=0) + 1.0
        dinv = lax.rsqrt(deg)
        o_ref[...] = dinv[:, None] * (a_ref[0] + a_ref[1] + g_ref[...]) + b_ref[...]

    out = pl.pallas_call(
        out_body,
        grid=(nb,),
        in_specs=[
            pl.BlockSpec((NC, BN, F_out), lambda i: (0, i, 0)),
            pl.BlockSpec((BN, F_out), lambda i: (i, 0)),
            pl.BlockSpec((NW, BN), lambda i: (0, i)),
            pl.BlockSpec((1, F_out), lambda i: (0, 0)),
        ],
        out_specs=pl.BlockSpec((BN, F_out), lambda i: (i, 0)),
        out_shape=jax.ShapeDtypeStruct((n_pad, F_out), jnp.float32),
    )(accs, g, degp, b.reshape(1, F_out))
    return out[:N]
